# baseline (speedup 1.0000x reference)
import functools

import jax
import jax.numpy as jnp
from jax import lax
from jax.experimental import pallas as pl
from jax.experimental.pallas import tpu as pltpu
from jax.experimental.pallas import tpu_sc as plsc

EMBED_DIM = 32
BATCH = 16384
NUM_CORES = 2
NUM_SUBCORES = 16
NUM_WORKERS = NUM_CORES * NUM_SUBCORES
ROWS_PER_WORKER = BATCH // NUM_WORKERS
CHUNK = 128
NUM_CHUNKS = ROWS_PER_WORKER // CHUNK

_mesh = plsc.VectorSubcoreMesh(core_axis_name="c", subcore_axis_name="s")


@functools.partial(
    pl.kernel,
    mesh=_mesh,
    out_type=jax.ShapeDtypeStruct((BATCH, EMBED_DIM), jnp.float32),
    scratch_types=[
        pltpu.VMEM((NUM_CHUNKS, CHUNK), jnp.int32),
        pltpu.VMEM((ROWS_PER_WORKER, EMBED_DIM), jnp.float32),
        pltpu.SemaphoreType.DMA,
    ],
    compiler_params=pltpu.CompilerParams(use_tc_tiling_on_sc=False),
)
def _gather_sc(table_hbm, idx_hbm, out_hbm, idx_v, rows_v, sem):
    wid = lax.axis_index("s") * NUM_CORES + lax.axis_index("c")
    pltpu.sync_copy(idx_hbm.at[wid], idx_v)
    copies = [
        pltpu.async_copy(
            table_hbm.at[idx_v.at[j]],
            rows_v.at[pl.ds(j * CHUNK, CHUNK)],
            sem,
        )
        for j in range(NUM_CHUNKS)
    ]
    for cp in copies:
        cp.wait()
    pltpu.sync_copy(rows_v, out_hbm.at[pl.ds(wid * ROWS_PER_WORKER, ROWS_PER_WORKER)])


def kernel(x, y):
    idx = x.astype(jnp.int32).reshape(NUM_WORKERS, NUM_CHUNKS, CHUNK)
    return _gather_sc(y, idx)


# speedup vs baseline: 3.5735x; 3.5735x over previous
import functools

import jax
import jax.numpy as jnp
from jax import lax
from jax.experimental import pallas as pl
from jax.experimental.pallas import tpu as pltpu
from jax.experimental.pallas import tpu_sc as plsc

EMBED_DIM = 32
BATCH = 16384
NUM_CORES = 2
NUM_SUBCORES = 16
NUM_WORKERS = NUM_CORES * NUM_SUBCORES
ROWS_PER_WORKER = BATCH // NUM_WORKERS
WAVE = 16
NUM_WAVES = ROWS_PER_WORKER // WAVE
GROUP = 128

_mesh = plsc.VectorSubcoreMesh(core_axis_name="c", subcore_axis_name="s")


@functools.partial(
    pl.kernel,
    mesh=_mesh,
    out_type=jax.ShapeDtypeStruct((EMBED_DIM, BATCH), jnp.float32),
    scratch_types=[
        pltpu.VMEM((ROWS_PER_WORKER,), jnp.int32),
        pltpu.VMEM((WAVE, EMBED_DIM, 128), jnp.float32),
        pltpu.VMEM((EMBED_DIM, GROUP), jnp.float32),
        pltpu.SemaphoreType.DMA,
    ],
    compiler_params=pltpu.CompilerParams(
        use_tc_tiling_on_sc=True, needs_layout_passes=False
    ),
)
def _gather_sc(yt_hbm, x_hbm, out_hbm, idx_v, blk_v, acc_v, sem):
    wid = lax.axis_index("s") * NUM_CORES + lax.axis_index("c")
    base = wid * ROWS_PER_WORKER
    pltpu.sync_copy(x_hbm.at[pl.ds(base, ROWS_PER_WORKER)], idx_v)
    rows0 = lax.iota(jnp.int32, 16)

    def wave_body(w, _):
        vec = idx_v[pl.ds(w * WAVE, WAVE)]
        for l in range(WAVE):
            r = vec[l]
            c = pl.multiple_of((r >> 7) * 128, 128)
            pltpu.async_copy(yt_hbm.at[:, pl.ds(c, 128)], blk_v.at[l], sem)
        for l in range(WAVE):
            pltpu.make_async_copy(
                yt_hbm.at[:, pl.ds(0, 128)], blk_v.at[l], sem
            ).wait()
        for l in range(WAVE):
            r = vec[l]
            lane = jnp.broadcast_to(r & 127, (16,))
            col = jnp.broadcast_to((w * WAVE + l) % GROUP, (16,))
            v0 = plsc.load_gather(blk_v.at[l], [rows0, lane])
            v1 = plsc.load_gather(blk_v.at[l], [rows0 + 16, lane])
            plsc.store_scatter(acc_v, [rows0, col], v0)
            plsc.store_scatter(acc_v, [rows0 + 16, col], v1)

        @pl.when((w + 1) % (GROUP // WAVE) == 0)
        def _():
            g = (w * WAVE) // GROUP
            off = pl.multiple_of(base + g * GROUP, 128)
            pltpu.sync_copy(acc_v, out_hbm.at[:, pl.ds(off, GROUP)])

        return ()

    lax.fori_loop(0, NUM_WAVES, wave_body, ())


def kernel(x, y):
    return _gather_sc(y.T, x.astype(jnp.int32)).T


# speedup vs baseline: 4.1048x; 1.1487x over previous
import functools

import jax
import jax.numpy as jnp
from jax import lax
from jax.experimental import pallas as pl
from jax.experimental.pallas import tpu as pltpu
from jax.experimental.pallas import tpu_sc as plsc

EMBED_DIM = 32
BATCH = 16384
NUM_CORES = 2
NUM_SUBCORES = 16
NUM_WORKERS = NUM_CORES * NUM_SUBCORES
ROWS_PER_WORKER = BATCH // NUM_WORKERS
WAVE = 16
NUM_WAVES = ROWS_PER_WORKER // WAVE
GROUP = 128

_mesh = plsc.VectorSubcoreMesh(core_axis_name="c", subcore_axis_name="s")


@functools.partial(
    pl.kernel,
    mesh=_mesh,
    out_type=jax.ShapeDtypeStruct((EMBED_DIM, BATCH), jnp.float32),
    scratch_types=[
        pltpu.VMEM((ROWS_PER_WORKER,), jnp.int32),
        pltpu.VMEM((WAVE, EMBED_DIM, 128), jnp.float32),
        pltpu.VMEM((EMBED_DIM, GROUP), jnp.float32),
        pltpu.SemaphoreType.DMA((WAVE,)),
    ],
    compiler_params=pltpu.CompilerParams(
        use_tc_tiling_on_sc=True, needs_layout_passes=False
    ),
)
def _gather_sc(yt_hbm, x_hbm, out_hbm, idx_v, blk_v, acc_v, sems):
    wid = lax.axis_index("s") * NUM_CORES + lax.axis_index("c")
    base = wid * ROWS_PER_WORKER
    pltpu.sync_copy(x_hbm.at[pl.ds(base, ROWS_PER_WORKER)], idx_v)
    rows0 = lax.iota(jnp.int32, 16)

    def fire(l, r):
        c = pl.multiple_of((r >> 7) * 128, 128)
        pltpu.async_copy(yt_hbm.at[:, pl.ds(c, 128)], blk_v.at[l], sems.at[l])

    vec0 = idx_v[pl.ds(0, WAVE)]
    for l in range(WAVE):
        fire(l, vec0[l])

    def wave_body(w, vec):
        nxt = jnp.minimum(w + 1, NUM_WAVES - 1)
        vec_next = idx_v[pl.ds(nxt * WAVE, WAVE)]
        for l in range(WAVE):
            pltpu.make_async_copy(
                yt_hbm.at[:, pl.ds(0, 128)], blk_v.at[l], sems.at[l]
            ).wait()
            r = vec[l]
            lane = jnp.broadcast_to(r & 127, (16,))
            col = jnp.broadcast_to((w * WAVE + l) % GROUP, (16,))
            v0 = plsc.load_gather(blk_v.at[l], [rows0, lane])
            v1 = plsc.load_gather(blk_v.at[l], [rows0 + 16, lane])
            plsc.store_scatter(acc_v, [rows0, col], v0)
            plsc.store_scatter(acc_v, [rows0 + 16, col], v1)

            @pl.when(w + 1 < NUM_WAVES)
            def _():
                fire(l, vec_next[l])

        @pl.when((w + 1) % (GROUP // WAVE) == 0)
        def _():
            g = (w * WAVE) // GROUP
            off = pl.multiple_of(base + g * GROUP, 128)
            pltpu.sync_copy(acc_v, out_hbm.at[:, pl.ds(off, GROUP)])

        return vec_next

    lax.fori_loop(0, NUM_WAVES, wave_body, vec0)


def kernel(x, y):
    return _gather_sc(y.T, x.astype(jnp.int32)).T
